# baseline (speedup 1.0000x reference)
import functools

import jax
import jax.numpy as jnp
from jax import lax
from jax.experimental import pallas as pl
from jax.experimental.pallas import tpu as pltpu
from jax.experimental.pallas import tpu_sc as plsc

BATCH = 16384
DIM = 64
NC = 2
NS = 16
NW = NC * NS
B_PER_W = BATCH // NW
CHUNK = 128
NCHUNK = B_PER_W // CHUNK

_mesh = plsc.VectorSubcoreMesh(core_axis_name="c", subcore_axis_name="s")


@functools.partial(
    pl.kernel,
    mesh=_mesh,
    out_type=jax.ShapeDtypeStruct((BATCH, DIM), jnp.float32),
    scratch_types=[
        pltpu.VMEM((NCHUNK, CHUNK), jnp.int32),
        pltpu.VMEM((B_PER_W, DIM), jnp.float32),
        pltpu.SemaphoreType.DMA,
    ],
    compiler_params=pltpu.CompilerParams(use_tc_tiling_on_sc=False),
)
def _gather_kernel(idx_hbm, table_hbm, out_hbm, idx_v, rows_v, sem):
    wid = lax.axis_index("s") * NC + lax.axis_index("c")
    pltpu.sync_copy(idx_hbm.at[wid], idx_v)
    copies = []
    for c in range(NCHUNK):
        copies.append(
            pltpu.async_copy(
                table_hbm.at[idx_v.at[c]],
                rows_v.at[pl.ds(c * CHUNK, CHUNK)],
                sem,
            )
        )
    for cp in copies:
        cp.wait()
    pltpu.sync_copy(rows_v, out_hbm.at[pl.ds(wid * B_PER_W, B_PER_W)])


def kernel(user_ids, long_pref_emb):
    idx = user_ids.astype(jnp.int32).reshape(NW, NCHUNK, CHUNK)
    return _gather_kernel(idx, long_pref_emb)


# speedup vs baseline: 1.7199x; 1.7199x over previous
import functools

import jax
import jax.numpy as jnp
from jax import lax
from jax.experimental import pallas as pl
from jax.experimental.pallas import tpu as pltpu
from jax.experimental.pallas import tpu_sc as plsc

BATCH = 16384
DIM = 64
NC = 2
NS = 16
NW = NC * NS
B_PER_W = BATCH // NW

_mesh = plsc.VectorSubcoreMesh(core_axis_name="c", subcore_axis_name="s")


@functools.partial(
    pl.kernel,
    mesh=_mesh,
    out_type=jax.ShapeDtypeStruct((BATCH, DIM), jnp.float32),
    scratch_types=[
        pltpu.VMEM((B_PER_W,), jnp.int32),
        pltpu.VMEM((B_PER_W, DIM), jnp.float32),
        pltpu.SemaphoreType.DMA,
    ],
)
def _gather_kernel(idx_hbm, table_hbm, out_hbm, idx_v, rows_v, sem):
    wid = lax.axis_index("s") * NC + lax.axis_index("c")
    base = wid * B_PER_W
    pltpu.sync_copy(idx_hbm.at[pl.ds(base, B_PER_W)], idx_v)

    def body(j, carry):
        v = idx_v[pl.ds(j * 16, 16)]
        for lane in range(16):
            pltpu.async_copy(
                table_hbm.at[pl.ds(v[lane], 1)],
                rows_v.at[pl.ds(j * 16 + lane, 1)],
                sem,
            )
        return carry

    lax.fori_loop(0, B_PER_W // 16, body, 0)
    pltpu.make_async_copy(table_hbm.at[pl.ds(0, B_PER_W)], rows_v, sem).wait()
    pltpu.sync_copy(rows_v, out_hbm.at[pl.ds(base, B_PER_W)])


def kernel(user_ids, long_pref_emb):
    return _gather_kernel(user_ids.astype(jnp.int32), long_pref_emb)


# speedup vs baseline: 2.5456x; 1.4801x over previous
import functools

import jax
import jax.numpy as jnp
from jax import lax
from jax.experimental import pallas as pl
from jax.experimental.pallas import tpu as pltpu
from jax.experimental.pallas import tpu_sc as plsc

BATCH = 16384
DIM = 64
NC = 2
NS = 16
NW = NC * NS
B_PER_W = BATCH // NW
NSLOT = 8
BLK = 128

_mesh = plsc.VectorSubcoreMesh(core_axis_name="c", subcore_axis_name="s")


@functools.partial(
    pl.kernel,
    mesh=_mesh,
    out_type=jax.ShapeDtypeStruct((BATCH, DIM), jnp.float32),
    scratch_types=[
        pltpu.VMEM((B_PER_W,), jnp.int32),
        pltpu.VMEM((NSLOT, DIM, BLK), jnp.float32),
        pltpu.VMEM((16, DIM), jnp.float32),
        pltpu.SemaphoreType.DMA,
    ],
    compiler_params=pltpu.CompilerParams(needs_layout_passes=False),
)
def _gather_kernel(idx_hbm, table_t_hbm, out_hbm, idx_v, ring_v, outblk_v, sem):
    wid = lax.axis_index("s") * NC + lax.axis_index("c")
    base = wid * B_PER_W
    pltpu.sync_copy(idx_hbm.at[pl.ds(base, B_PER_W)], idx_v)
    lanes = lax.iota(jnp.int32, 16)

    def fire(r, slot):
        b0 = pl.multiple_of((r >> 7) << 7, BLK)
        pltpu.async_copy(
            table_t_hbm.at[:, pl.ds(b0, BLK)], ring_v.at[slot], sem
        )

    def wait_one():
        pltpu.make_async_copy(
            table_t_hbm.at[:, pl.ds(0, BLK)], ring_v.at[0], sem
        ).wait()

    def extract(r, slot, u):
        c = r & (BLK - 1)
        row = outblk_v.at[u]
        for g in range(DIM // 16):
            feat = lanes + (16 * g)
            vals = plsc.load_gather(
                ring_v,
                [jnp.full((16,), slot, jnp.int32), feat,
                 jnp.full((16,), c, jnp.int32)],
            )
            row[pl.ds(16 * g, 16)] = vals

    def body2(j, carry):
        v = idx_v[pl.ds(j * 16, 16)]
        for lane in range(8):
            fire(v[lane], lane)
        for lane in range(8):
            wait_one()
            extract(v[lane], lane, lane)
        for lane in range(8):
            fire(v[8 + lane], lane)
        for lane in range(8):
            wait_one()
            extract(v[8 + lane], lane, 8 + lane)
        pltpu.sync_copy(outblk_v, out_hbm.at[pl.ds(base + j * 16, 16)])
        return carry

    lax.fori_loop(0, B_PER_W // 16, body2, 0)


def kernel(user_ids, long_pref_emb):
    return _gather_kernel(user_ids.astype(jnp.int32), long_pref_emb.T)


# speedup vs baseline: 3.2299x; 1.2688x over previous
import functools

import jax
import jax.numpy as jnp
from jax import lax
from jax.experimental import pallas as pl
from jax.experimental.pallas import tpu as pltpu
from jax.experimental.pallas import tpu_sc as plsc

BATCH = 16384
DIM = 64
NUSERS = 1000000
NC = 2
NS = 16
NW = NC * NS
CHW = 256
NQ = (NUSERS + CHW - 1) // CHW
Q_PER_W = (NQ + NW - 1) // NW
TAILQ = NQ - 1
TAILW = NUSERS - TAILQ * CHW

_mesh = plsc.VectorSubcoreMesh(core_axis_name="c", subcore_axis_name="s")


@functools.partial(
    pl.kernel,
    mesh=_mesh,
    out_type=jax.ShapeDtypeStruct((BATCH, DIM), jnp.float32),
    scratch_types=[
        pltpu.VMEM((BATCH,), jnp.int32),
        pltpu.VMEM((BATCH + 16,), jnp.int32),
        pltpu.VMEM((BATCH + 16,), jnp.int32),
        pltpu.VMEM((2, DIM, CHW), jnp.float32),
        pltpu.VMEM((272,), jnp.int32),
        pltpu.VMEM((272,), jnp.int32),
        pltpu.VMEM((16, DIM), jnp.float32),
        pltpu.VMEM((TAILW, DIM), jnp.float32),
        pltpu.SemaphoreType.DMA,
        pltpu.SemaphoreType.DMA,
    ],
    compiler_params=pltpu.CompilerParams(needs_layout_passes=False),
)
def _gather_kernel(idx_hbm, table_t_hbm, tail_hbm, out_hbm, ids_v, hid_v,
                   hpos_v, chunk_v, cbuf_v, pbuf_v, stage_v, tail_v,
                   sem_c, sem_o):
    wid = lax.axis_index("s") * NC + lax.axis_index("c")
    lanes = lax.iota(jnp.int32, 16)
    q_base = wid * Q_PER_W
    lo = q_base * CHW
    hi = jnp.minimum(lo + Q_PER_W * CHW, NUSERS)

    pltpu.sync_copy(idx_hbm, ids_v)
    pltpu.sync_copy(tail_hbm, tail_v)

    def scan_body(t, off):
        v = ids_v[pl.ds(t * 16, 16)]
        m = (v >= lo) & (v < hi)
        plsc.store_compressed(hid_v.at[pl.ds(off, 16)], v, mask=m)
        plsc.store_compressed(hpos_v.at[pl.ds(off, 16)], t * 16 + lanes, mask=m)
        return off + plsc.all_reduce_population_count(m)[0]

    nh = lax.fori_loop(0, BATCH // 16, scan_body, jnp.int32(0))

    def fire(k):
        q = q_base + k

        @pl.when(q < TAILQ)
        def _():
            g0 = pl.multiple_of(q * CHW, 128)
            pltpu.async_copy(
                table_t_hbm.at[:, pl.ds(g0, CHW)], chunk_v.at[k & 1], sem_c
            )

    def wait_chunk(k):
        q = q_base + k

        @pl.when(q < TAILQ)
        def _():
            pltpu.make_async_copy(
                table_t_hbm.at[:, pl.ds(0, CHW)], chunk_v.at[0], sem_c
            ).wait()

    def drain_out_rows(n):
        def d(_, carry):
            pltpu.make_async_copy(
                stage_v.at[pl.ds(0, 1)], out_hbm.at[pl.ds(0, 1)], sem_o
            ).wait()
            return carry

        lax.fori_loop(0, n, d, jnp.int32(0))

    def extract_vec(e, nc, k):
        rem = nc - e * 16
        cv = cbuf_v[pl.ds(e * 16, 16)]
        pv = pbuf_v[pl.ds(e * 16, 16)]
        for lane in range(16):
            @pl.when(lane < rem)
            def _():
                c = cv[lane]
                p = pv[lane]
                row = stage_v.at[lane]
                for g in range(DIM // 16):
                    vals = plsc.load_gather(
                        chunk_v,
                        [jnp.full((16,), k & 1, jnp.int32),
                         lanes + (16 * g),
                         jnp.full((16,), c, jnp.int32)],
                    )
                    row[pl.ds(16 * g, 16)] = vals
                pltpu.async_copy(
                    stage_v.at[pl.ds(lane, 1)], out_hbm.at[pl.ds(p, 1)], sem_o
                )
        drain_out_rows(jnp.minimum(rem, 16))

    def extract_tail_vec(e, nc):
        rem = nc - e * 16
        cv = cbuf_v[pl.ds(e * 16, 16)]
        pv = pbuf_v[pl.ds(e * 16, 16)]
        for lane in range(16):
            @pl.when(lane < rem)
            def _():
                c = cv[lane]
                p = pv[lane]
                pltpu.async_copy(
                    tail_v.at[pl.ds(c, 1)], out_hbm.at[pl.ds(p, 1)], sem_o
                )
        drain_out_rows(jnp.minimum(rem, 16))

    def chunk_body(k, carry):
        @pl.when(k + 1 < Q_PER_W)
        def _():
            fire(k + 1)

        wait_chunk(k)
        q = q_base + k

        @pl.when(q < NQ)
        def _():
            g0 = q * CHW

            def batch_body(t2, carry2):
                def gather_hits(t, off):
                    hv = hid_v[pl.ds(t * 16, 16)]
                    pvv = hpos_v[pl.ds(t * 16, 16)]
                    valid = (t * 16 + lanes) < nh
                    m = valid & (hv >= g0) & (hv < g0 + CHW)
                    plsc.store_compressed(
                        cbuf_v.at[pl.ds(off, 16)], hv - g0, mask=m)
                    plsc.store_compressed(
                        pbuf_v.at[pl.ds(off, 16)], pvv, mask=m)
                    return off + plsc.all_reduce_population_count(m)[0]

                nc = lax.fori_loop(
                    t2 * 16, jnp.minimum(t2 * 16 + 16, (nh + 15) // 16),
                    gather_hits, jnp.int32(0))

                def ex(e, c3):
                    @pl.when(q < TAILQ)
                    def _():
                        extract_vec(e, nc, k)

                    @pl.when(q == TAILQ)
                    def _():
                        extract_tail_vec(e, nc)

                    return c3

                lax.fori_loop(0, (nc + 15) // 16, ex, jnp.int32(0))
                return carry2

            lax.fori_loop(0, (nh + 255) // 256, batch_body, jnp.int32(0))

        return carry

    fire(0)
    lax.fori_loop(0, Q_PER_W, chunk_body, jnp.int32(0))


def kernel(user_ids, long_pref_emb):
    tail = long_pref_emb[NUSERS - TAILW:]
    return _gather_kernel(user_ids.astype(jnp.int32), long_pref_emb.T, tail)
